# Initial kernel scaffold; baseline (speedup 1.0000x reference)
import jax
import jax.numpy as jnp
from jax.experimental import pallas as pl


def kernel(in_feat, edge_index, W1, b1, W2, b2):
    raise NotImplementedError("write your pallas kernel here")


# speedup vs baseline: 5.4952x; 5.4952x over previous
import functools

import jax
import jax.numpy as jnp
from jax import lax
from jax.experimental import pallas as pl
from jax.experimental.pallas import tpu as pltpu
from jax.experimental.pallas import tpu_sc as plsc

_N = 10000
_E = 320000
_D = 128
_DOUT = 40

_NC = 2
_NS = 16
_NW = _NC * _NS
_K = 128
_CHUNKS = 79
_EPW = _CHUNKS * _K
_EPAD = _NW * _EPW
_NPAD = 10240
_RPT = _NPAD // _NS
_BM = 256
_NBLK = _NPAD // _BM

_f32 = jnp.float32
_i32 = jnp.int32


def _sc_mesh():
    return plsc.VectorSubcoreMesh(
        core_axis_name="c", subcore_axis_name="s",
        num_cores=_NC, num_subcores=_NS)


def _hist_body(src_hbm, dst_hbm, out_hbm, sidx, didx, ones, zvec, ho_sh, hi_sh):
    cid = lax.axis_index("c")
    sid = lax.axis_index("s")
    wid = sid * _NC + cid
    z16 = jnp.zeros((16,), _f32)
    o16 = jnp.ones((16,), _f32)
    for j in range(_K // 16):
        ones[pl.ds(j * 16, 16)] = o16
    for j in range(_RPT // 16):
        zvec[pl.ds(j * 16, 16)] = z16
    pltpu.sync_copy(zvec, ho_sh.at[pl.ds(sid * _RPT, _RPT)])
    pltpu.sync_copy(zvec, hi_sh.at[pl.ds(sid * _RPT, _RPT)])
    plsc.subcore_barrier()

    def chunk(g, carry):
        base = pl.multiple_of(wid * _EPW + g * _K, _K)
        pltpu.sync_copy(src_hbm.at[pl.ds(base, _K)], sidx)
        pltpu.sync_copy(dst_hbm.at[pl.ds(base, _K)], didx)
        pltpu.sync_copy(ones, ho_sh.at[sidx], add=True)
        pltpu.sync_copy(ones, hi_sh.at[didx], add=True)
        return carry

    lax.fori_loop(0, _CHUNKS, chunk, 0)
    plsc.subcore_barrier()
    pltpu.sync_copy(ho_sh.at[pl.ds(sid * _RPT, _RPT)],
                    out_hbm.at[pl.ds((cid * 2 + 0) * _NPAD + sid * _RPT, _RPT)])
    pltpu.sync_copy(hi_sh.at[pl.ds(sid * _RPT, _RPT)],
                    out_hbm.at[pl.ds((cid * 2 + 1) * _NPAD + sid * _RPT, _RPT)])


def _run_hist(srcp, dstp):
    k = pl.kernel(
        _hist_body,
        out_type=jax.ShapeDtypeStruct((4 * _NPAD,), _f32),
        mesh=_sc_mesh(),
        scratch_types=[
            pltpu.VMEM((_K,), _i32),
            pltpu.VMEM((_K,), _i32),
            pltpu.VMEM((_K,), _f32),
            pltpu.VMEM((_RPT,), _f32),
            pltpu.VMEM_SHARED((_NPAD,), _f32),
            pltpu.VMEM_SHARED((_NPAD,), _f32),
        ],
    )
    return k(srcp, dstp)


def _pre_body(x_ref, w1_ref, degp_ref, h0s_ref, ns_ref, nd_ref):
    dego = degp_ref[0, 0] + degp_ref[1, 0]
    degi = degp_ref[0, 1] + degp_ref[1, 1]
    ns = lax.rsqrt(jnp.maximum(dego, 1.0))
    nd = lax.rsqrt(jnp.maximum(degi, 1.0))
    ns_ref[...] = ns
    nd_ref[...] = nd
    h0s_ref[...] = jnp.dot(x_ref[...], w1_ref[...],
                           preferred_element_type=_f32) * ns


def _run_pre(xp, W1, degp4):
    return pl.pallas_call(
        _pre_body,
        grid=(_NBLK,),
        in_specs=[
            pl.BlockSpec((_BM, _D), lambda b: (b, 0)),
            pl.BlockSpec((_D, _D), lambda b: (0, 0)),
            pl.BlockSpec((2, 2, _BM, 1), lambda b: (0, 0, b, 0)),
        ],
        out_specs=[
            pl.BlockSpec((_BM, _D), lambda b: (b, 0)),
            pl.BlockSpec((_BM, 1), lambda b: (b, 0)),
            pl.BlockSpec((_BM, 1), lambda b: (b, 0)),
        ],
        out_shape=[
            jax.ShapeDtypeStruct((_NPAD, _D), _f32),
            jax.ShapeDtypeStruct((_NPAD, 1), _f32),
            jax.ShapeDtypeStruct((_NPAD, 1), _f32),
        ],
    )(xp, W1, degp4)


def _main_body(h0s_hbm, src_hbm, dst_hbm, nd_hbm, aggp_hbm, wp_hbm,
               sidx, didx, rows, wvals, gsem, wsem,
               agg_sh, w_sh):
    cid = lax.axis_index("c")
    sid = lax.axis_index("s")
    wid = sid * _NC + cid
    z16 = jnp.zeros((16,), _f32)

    def zr(r, carry):
        for j in range(_D // 16):
            rows[r, pl.ds(j * 16, 16)] = z16
        return carry

    lax.fori_loop(0, _K, zr, 0)
    for j in range(_K // 16):
        wvals[pl.ds(j * 16, 16)] = z16
    for h in range(_RPT // _K):
        pltpu.sync_copy(rows, agg_sh.at[pl.ds(sid * _RPT + h * _K, _K)])
        pltpu.sync_copy(wvals, w_sh.at[pl.ds(sid * _RPT + h * _K, _K)])
    plsc.subcore_barrier()

    def chunk(g, carry):
        base = pl.multiple_of(wid * _EPW + g * _K, _K)
        pltpu.sync_copy(src_hbm.at[pl.ds(base, _K)], sidx)
        pltpu.sync_copy(dst_hbm.at[pl.ds(base, _K)], didx)
        pltpu.async_copy(h0s_hbm.at[sidx], rows, gsem).wait()
        pltpu.sync_copy(rows, agg_sh.at[didx], add=True)
        pltpu.async_copy(nd_hbm.at[didx], wvals, wsem).wait()
        pltpu.sync_copy(wvals, w_sh.at[sidx], add=True)
        return carry

    lax.fori_loop(0, _CHUNKS, chunk, 0)
    plsc.subcore_barrier()
    pltpu.sync_copy(agg_sh.at[pl.ds(sid * _RPT, _RPT)],
                    aggp_hbm.at[pl.ds(cid * _NPAD + sid * _RPT, _RPT)])
    pltpu.sync_copy(w_sh.at[pl.ds(sid * _RPT, _RPT)],
                    wp_hbm.at[pl.ds(cid * _NPAD + sid * _RPT, _RPT)])


def _run_main(h0s, srcp, dstp, nd_flat):
    k = pl.kernel(
        _main_body,
        out_type=(
            jax.ShapeDtypeStruct((2 * _NPAD, _D), _f32),
            jax.ShapeDtypeStruct((2 * _NPAD,), _f32),
        ),
        mesh=_sc_mesh(),
        scratch_types=[
            pltpu.VMEM((_K,), _i32),
            pltpu.VMEM((_K,), _i32),
            pltpu.VMEM((_K, _D), _f32),
            pltpu.VMEM((_K,), _f32),
            pltpu.SemaphoreType.DMA,
            pltpu.SemaphoreType.DMA,
            pltpu.VMEM_SHARED((_NPAD, _D), _f32),
            pltpu.VMEM_SHARED((_NPAD,), _f32),
        ],
    )
    return k(h0s, srcp, dstp, nd_flat)


def _post_body(aggp_ref, wp_ref, ns_ref, nd_ref, b1_ref, w2_ref, b2_ref,
               out_ref, acc):
    b = pl.program_id(0)

    @pl.when(b == 0)
    def _():
        acc[...] = jnp.zeros_like(acc)

    agg = aggp_ref[0] + aggp_ref[1]
    h1 = jnp.maximum(agg * nd_ref[...] + b1_ref[...], 0.0)
    w = wp_ref[0] + wp_ref[1]
    c = ns_ref[...] * w
    rid = b * _BM + lax.broadcasted_iota(_i32, (_BM, 1), 0)
    c = jnp.where(rid < _N, c, 0.0)
    acc[...] += jnp.sum(h1 * c, axis=0, keepdims=True)

    @pl.when(b == _NBLK - 1)
    def _():
        out_ref[...] = (jnp.dot(acc[...], w2_ref[...],
                                preferred_element_type=_f32) * (1.0 / _N)
                        + b2_ref[...])


def _run_post(aggp, wp, ns, nd, b1, W2, b2):
    return pl.pallas_call(
        _post_body,
        grid=(_NBLK,),
        in_specs=[
            pl.BlockSpec((2, _BM, _D), lambda b: (0, b, 0)),
            pl.BlockSpec((2, _BM, 1), lambda b: (0, b, 0)),
            pl.BlockSpec((_BM, 1), lambda b: (b, 0)),
            pl.BlockSpec((_BM, 1), lambda b: (b, 0)),
            pl.BlockSpec((1, _D), lambda b: (0, 0)),
            pl.BlockSpec((_D, _DOUT), lambda b: (0, 0)),
            pl.BlockSpec((1, _DOUT), lambda b: (0, 0)),
        ],
        out_specs=pl.BlockSpec((1, _DOUT), lambda b: (0, 0)),
        out_shape=jax.ShapeDtypeStruct((1, _DOUT), _f32),
        scratch_shapes=[pltpu.VMEM((1, _D), _f32)],
    )(aggp, wp, ns, nd, b1, W2, b2)


def kernel(in_feat, edge_index, W1, b1, W2, b2):
    src = edge_index[0]
    dst = edge_index[1]
    pad = _EPAD - _E
    fill = jnp.full((pad,), _N, _i32)
    srcp = jnp.concatenate([src, fill])
    dstp = jnp.concatenate([dst, fill])
    xp = jnp.pad(in_feat, ((0, _NPAD - _N), (0, 0)))

    degp = _run_hist(srcp, dstp)
    degp4 = degp.reshape(2, 2, _NPAD, 1)
    h0s, ns, nd = _run_pre(xp, W1, degp4)
    aggp, wp = _run_main(h0s, srcp, dstp, nd.reshape(_NPAD))
    out = _run_post(aggp.reshape(2, _NPAD, _D), wp.reshape(2, _NPAD, 1),
                    ns, nd, b1.reshape(1, _D), W2, b2.reshape(1, _DOUT))
    return out


# speedup vs baseline: 5.6672x; 1.0313x over previous
import functools

import jax
import jax.numpy as jnp
from jax import lax
from jax.experimental import pallas as pl
from jax.experimental.pallas import tpu as pltpu
from jax.experimental.pallas import tpu_sc as plsc

_N = 10000
_E = 320000
_D = 128
_DOUT = 40

_NC = 2
_NS = 16
_NW = _NC * _NS
_K = 128
_CHUNKS = 80
_HALF = _CHUNKS // 2
_EPW = _CHUNKS * _K
_EPAD = _NW * _EPW
_NPAD = 10240
_RPT = _NPAD // _NS
_BM = 256
_NBLK = _NPAD // _BM

_f32 = jnp.float32
_i32 = jnp.int32


def _sc_mesh():
    return plsc.VectorSubcoreMesh(
        core_axis_name="c", subcore_axis_name="s",
        num_cores=_NC, num_subcores=_NS)


def _hist_body(src_hbm, dst_hbm, out_hbm, sidxs, didxs, ones, zvec, asem,
               ho_sh, hi_sh):
    cid = lax.axis_index("c")
    sid = lax.axis_index("s")
    wid = sid * _NC + cid
    z16 = jnp.zeros((16,), _f32)
    o16 = jnp.ones((16,), _f32)
    for j in range(_K // 16):
        ones[pl.ds(j * 16, 16)] = o16
        zvec[pl.ds(j * 16, 16)] = z16
    for i in range(_RPT // _K):
        pltpu.sync_copy(zvec, ho_sh.at[pl.ds(sid * _RPT + i * _K, _K)])
        pltpu.sync_copy(zvec, hi_sh.at[pl.ds(sid * _RPT + i * _K, _K)])
    pltpu.sync_copy(src_hbm.at[wid], sidxs)
    pltpu.sync_copy(dst_hbm.at[wid], didxs)
    plsc.subcore_barrier()

    def fire(c):
        pltpu.async_copy(ones, ho_sh.at[sidxs.at[c]], asem, add=True)
        pltpu.async_copy(ones, hi_sh.at[didxs.at[c]], asem, add=True)

    def drain2():
        pltpu.make_async_copy(out_hbm.at[pl.ds(0, _K)], ones, asem).wait()
        pltpu.make_async_copy(out_hbm.at[pl.ds(0, _K)], ones, asem).wait()

    fire(0)
    fire(1)

    def chunk(c, carry):
        drain2()

        @pl.when(c + 2 < _CHUNKS)
        def _():
            fire(c + 2)

        return carry

    lax.fori_loop(0, _CHUNKS, chunk, 0)
    plsc.subcore_barrier()
    pltpu.sync_copy(ho_sh.at[pl.ds(sid * _RPT, _RPT)],
                    out_hbm.at[pl.ds((cid * 2 + 0) * _NPAD + sid * _RPT, _RPT)])
    pltpu.sync_copy(hi_sh.at[pl.ds(sid * _RPT, _RPT)],
                    out_hbm.at[pl.ds((cid * 2 + 1) * _NPAD + sid * _RPT, _RPT)])


def _run_hist(src3, dst3):
    k = pl.kernel(
        _hist_body,
        out_type=jax.ShapeDtypeStruct((4 * _NPAD,), _f32),
        mesh=_sc_mesh(),
        scratch_types=[
            pltpu.VMEM((_CHUNKS, _K), _i32),
            pltpu.VMEM((_CHUNKS, _K), _i32),
            pltpu.VMEM((_K,), _f32),
            pltpu.VMEM((_K,), _f32),
            pltpu.SemaphoreType.DMA,
            pltpu.VMEM_SHARED((_NPAD,), _f32),
            pltpu.VMEM_SHARED((_NPAD,), _f32),
        ],
    )
    return k(src3, dst3)


def _pre_body(x_ref, w1_ref, degp_ref, h0s_ref, ns_ref, nd_ref):
    dego = degp_ref[0, 0] + degp_ref[1, 0]
    degi = degp_ref[0, 1] + degp_ref[1, 1]
    ns = lax.rsqrt(jnp.maximum(dego, 1.0))
    nd = lax.rsqrt(jnp.maximum(degi, 1.0))
    ns_ref[...] = ns
    nd_ref[...] = nd
    h0s_ref[...] = jnp.dot(x_ref[...], w1_ref[...],
                           preferred_element_type=_f32) * ns


def _run_pre(xp, W1, degp4):
    return pl.pallas_call(
        _pre_body,
        grid=(_NBLK,),
        in_specs=[
            pl.BlockSpec((_BM, _D), lambda b: (b, 0)),
            pl.BlockSpec((_D, _D), lambda b: (0, 0)),
            pl.BlockSpec((2, 2, _BM, 1), lambda b: (0, 0, b, 0)),
        ],
        out_specs=[
            pl.BlockSpec((_BM, _D), lambda b: (b, 0)),
            pl.BlockSpec((_BM, 1), lambda b: (b, 0)),
            pl.BlockSpec((_BM, 1), lambda b: (b, 0)),
        ],
        out_shape=[
            jax.ShapeDtypeStruct((_NPAD, _D), _f32),
            jax.ShapeDtypeStruct((_NPAD, 1), _f32),
            jax.ShapeDtypeStruct((_NPAD, 1), _f32),
        ],
    )(xp, W1, degp4)


def _main_body(h0s_hbm, src_hbm, dst_hbm, nd_hbm, aggp_hbm, wp_hbm,
               sidxs, didxs, rows, wval, gsem0, gsem1,
               agg_sh, w_sh):
    cid = lax.axis_index("c")
    sid = lax.axis_index("s")
    wid = sid * _NC + cid
    z16 = jnp.zeros((16,), _f32)

    def zr(r, carry):
        for j in range(_D // 16):
            rows[r, pl.ds(j * 16, 16)] = z16
        return carry

    lax.fori_loop(0, _K, zr, 0)
    for j in range(_K // 16):
        wval[0, pl.ds(j * 16, 16)] = z16
    zrows = rows.at[pl.ds(0, _K)]
    for i in range(_RPT // _K):
        pltpu.sync_copy(zrows, agg_sh.at[pl.ds(sid * _RPT + i * _K, _K)])
        pltpu.sync_copy(wval.at[0], w_sh.at[pl.ds(sid * _RPT + i * _K, _K)])
    plsc.subcore_barrier()

    gsems = (gsem0, gsem1)

    for h in range(2):
        pltpu.sync_copy(src_hbm.at[wid, pl.ds(h * _HALF, _HALF)], sidxs)
        pltpu.sync_copy(dst_hbm.at[wid, pl.ds(h * _HALF, _HALF)], didxs)

        def fire(c, p):
            pltpu.async_copy(h0s_hbm.at[sidxs.at[c]],
                             rows.at[pl.ds(p * _K, _K)], gsems[p])
            pltpu.async_copy(nd_hbm.at[didxs.at[c]], wval.at[c], gsems[p])

        def drain(p):
            pltpu.make_async_copy(h0s_hbm.at[pl.ds(0, _K)],
                                  rows.at[pl.ds(p * _K, _K)], gsems[p]).wait()
            pltpu.make_async_copy(nd_hbm.at[pl.ds(0, _K)],
                                  wval.at[0], gsems[p]).wait()

        def scat(c, p):
            pltpu.sync_copy(rows.at[pl.ds(p * _K, _K)],
                            agg_sh.at[didxs.at[c]], add=True)
            pltpu.sync_copy(wval.at[c], w_sh.at[sidxs.at[c]], add=True)

        fire(0, 0)
        fire(1, 1)

        def step(u, carry):
            for p in range(2):
                c = 2 * u + p
                drain(p)
                scat(c, p)

                @pl.when(c + 2 < _HALF)
                def _():
                    fire(c + 2, p)

            return carry

        lax.fori_loop(0, _HALF // 2, step, 0)

    plsc.subcore_barrier()
    pltpu.sync_copy(agg_sh.at[pl.ds(sid * _RPT, _RPT)],
                    aggp_hbm.at[pl.ds(cid * _NPAD + sid * _RPT, _RPT)])
    pltpu.sync_copy(w_sh.at[pl.ds(sid * _RPT, _RPT)],
                    wp_hbm.at[pl.ds(cid * _NPAD + sid * _RPT, _RPT)])


def _run_main(h0s, src3, dst3, nd_flat):
    k = pl.kernel(
        _main_body,
        out_type=(
            jax.ShapeDtypeStruct((2 * _NPAD, _D), _f32),
            jax.ShapeDtypeStruct((2 * _NPAD,), _f32),
        ),
        mesh=_sc_mesh(),
        scratch_types=[
            pltpu.VMEM((_HALF, _K), _i32),
            pltpu.VMEM((_HALF, _K), _i32),
            pltpu.VMEM((2 * _K, _D), _f32),
            pltpu.VMEM((_HALF, _K), _f32),
            pltpu.SemaphoreType.DMA,
            pltpu.SemaphoreType.DMA,
            pltpu.VMEM_SHARED((_NPAD, _D), _f32),
            pltpu.VMEM_SHARED((_NPAD,), _f32),
        ],
    )
    return k(h0s, src3, dst3, nd_flat)


def _post_body(aggp_ref, wp_ref, ns_ref, nd_ref, b1_ref, w2_ref, b2_ref,
               out_ref, acc):
    b = pl.program_id(0)

    @pl.when(b == 0)
    def _():
        acc[...] = jnp.zeros_like(acc)

    agg = aggp_ref[0] + aggp_ref[1]
    h1 = jnp.maximum(agg * nd_ref[...] + b1_ref[...], 0.0)
    w = wp_ref[0] + wp_ref[1]
    c = ns_ref[...] * w
    rid = b * _BM + lax.broadcasted_iota(_i32, (_BM, 1), 0)
    c = jnp.where(rid < _N, c, 0.0)
    acc[...] += jnp.sum(h1 * c, axis=0, keepdims=True)

    @pl.when(b == _NBLK - 1)
    def _():
        out_ref[...] = (jnp.dot(acc[...], w2_ref[...],
                                preferred_element_type=_f32) * (1.0 / _N)
                        + b2_ref[...])


def _run_post(aggp, wp, ns, nd, b1, W2, b2):
    return pl.pallas_call(
        _post_body,
        grid=(_NBLK,),
        in_specs=[
            pl.BlockSpec((2, _BM, _D), lambda b: (0, b, 0)),
            pl.BlockSpec((2, _BM, 1), lambda b: (0, b, 0)),
            pl.BlockSpec((_BM, 1), lambda b: (b, 0)),
            pl.BlockSpec((_BM, 1), lambda b: (b, 0)),
            pl.BlockSpec((1, _D), lambda b: (0, 0)),
            pl.BlockSpec((_D, _DOUT), lambda b: (0, 0)),
            pl.BlockSpec((1, _DOUT), lambda b: (0, 0)),
        ],
        out_specs=pl.BlockSpec((1, _DOUT), lambda b: (0, 0)),
        out_shape=jax.ShapeDtypeStruct((1, _DOUT), _f32),
        scratch_shapes=[pltpu.VMEM((1, _D), _f32)],
    )(aggp, wp, ns, nd, b1, W2, b2)


def kernel(in_feat, edge_index, W1, b1, W2, b2):
    src = edge_index[0]
    dst = edge_index[1]
    pad = _EPAD - _E
    fill = jnp.full((pad,), _N, _i32)
    src3 = jnp.concatenate([src, fill]).reshape(_NW, _CHUNKS, _K)
    dst3 = jnp.concatenate([dst, fill]).reshape(_NW, _CHUNKS, _K)
    xp = jnp.pad(in_feat, ((0, _NPAD - _N), (0, 0)))

    degp = _run_hist(src3, dst3)
    degp4 = degp.reshape(2, 2, _NPAD, 1)
    h0s, ns, nd = _run_pre(xp, W1, degp4)
    aggp, wp = _run_main(h0s, src3, dst3, nd.reshape(_NPAD))
    out = _run_post(aggp.reshape(2, _NPAD, _D), wp.reshape(2, _NPAD, 1),
                    ns, nd, b1.reshape(1, _D), W2, b2.reshape(1, _DOUT))
    return out
